# baseline (speedup 1.0000x reference)
import functools

import jax
import jax.numpy as jnp
from jax import lax
from jax.experimental import pallas as pl
from jax.experimental.pallas import tpu as pltpu
from jax.experimental.pallas import tpu_sc as plsc

_LANES = 16
_NTILES = 16
_NCORES = 2
_CH = 128


def _build_table(x_pad, W_rel, b_rel3):
    n_pad, H = x_pad.shape
    R = W_rel.shape[0]
    BLK = 1024
    nb = n_pad // BLK

    def body(x_ref, w_ref, b_ref, o_ref):
        mm = jnp.dot(x_ref[...], w_ref[0], preferred_element_type=jnp.float32)
        o_ref[0] = mm + b_ref[0, 0][None, :]

    out = pl.pallas_call(
        body,
        grid=(R, nb),
        in_specs=[
            pl.BlockSpec((BLK, H), lambda r, b: (b, 0)),
            pl.BlockSpec((1, H, H), lambda r, b: (r, 0, 0)),
            pl.BlockSpec((1, 1, H), lambda r, b: (r, 0, 0)),
        ],
        out_specs=pl.BlockSpec((1, BLK, H), lambda r, b: (r, b, 0)),
        out_shape=jax.ShapeDtypeStruct((R, n_pad, H), jnp.float32),
    )(x_pad, W_rel, b_rel3)
    return out.reshape(R * n_pad, H)


def _sc_edges(table, src2, dst128, et2, fi2, zrows):
    n_pad = fi2.shape[1]
    H = table.shape[1]
    e_pad = src2.shape[1]
    epw = e_pad // _NTILES
    SUP = 1024
    nsub = SUP // _CH
    nsuper = epw // SUP
    rpw = n_pad // _NTILES
    mesh = plsc.VectorSubcoreMesh(core_axis_name="c", subcore_axis_name="s")

    @functools.partial(
        pl.kernel,
        out_type=jax.ShapeDtypeStruct((_NCORES * n_pad, H), jnp.float32),
        mesh=mesh,
        compiler_params=pltpu.CompilerParams(needs_layout_passes=False),
        scratch_types=[
            pltpu.VMEM_SHARED((n_pad, H), jnp.float32),
            pltpu.VMEM((n_pad,), jnp.int32),
            pltpu.VMEM((SUP,), jnp.int32),
            pltpu.VMEM((SUP,), jnp.int32),
            pltpu.VMEM((nsub, _CH), jnp.int32),
            pltpu.VMEM((nsub, _CH), jnp.int32),
            pltpu.VMEM((_CH, H), jnp.float32),
            pltpu.VMEM((_CH, H), jnp.float32),
            pltpu.SemaphoreType.DMA,
            pltpu.SemaphoreType.DMA,
            pltpu.SemaphoreType.DMA,
            pltpu.SemaphoreType.DMA,
        ],
    )
    def k(table_h, src_h, dst_h, et_h, fi_h, z_h,
          agg_h,
          acc, fi_v, src_c, et_c, dst_c, gix_c, rows0, rows1,
          gsem0, gsem1, ssem0, ssem1):
        c = lax.axis_index("c")
        s = lax.axis_index("s")
        pltpu.sync_copy(fi_h.at[c], fi_v)
        row0 = pl.multiple_of(s * rpw, 8)
        pltpu.sync_copy(z_h, acc.at[pl.ds(row0, rpw)])
        plsc.subcore_barrier()

        rbufs = (rows0, rows1)
        gsems = (gsem0, gsem1)
        ssems = (ssem0, ssem1)

        def super_body(kk, carry):
            base = pl.multiple_of(s * epw + kk * SUP, 8)
            base128 = pl.multiple_of((s * epw + kk * SUP) // _CH, 8)
            pltpu.sync_copy(src_h.at[c, pl.ds(base, SUP)], src_c)
            pltpu.sync_copy(et_h.at[c, pl.ds(base, SUP)], et_c)
            pltpu.sync_copy(dst_h.at[c, pl.ds(base128, nsub)], dst_c)
            for b in range(nsub):
                for g in range(_CH // _LANES):
                    sl = pl.ds(g * _LANES, _LANES)
                    sle = pl.ds(b * _CH + g * _LANES, _LANES)
                    f16 = plsc.load_gather(fi_v, [src_c[sle]])
                    gix_c[b, sl] = et_c[sle] * n_pad + f16
            gat = [None, None]
            scat = [None, None]
            gat[0] = pltpu.async_copy(table_h.at[gix_c.at[0]], rbufs[0],
                                      gsems[0])
            for b in range(nsub):
                i = b & 1
                if b + 1 < nsub:
                    j = (b + 1) & 1
                    if scat[j] is not None:
                        scat[j].wait()
                    gat[j] = pltpu.async_copy(
                        table_h.at[gix_c.at[b + 1]], rbufs[j], gsems[j])
                gat[i].wait()
                scat[i] = pltpu.async_copy(rbufs[i], acc.at[dst_c.at[b]],
                                           ssems[i], add=True)
            scat[0].wait()
            scat[1].wait()
            return carry

        lax.fori_loop(0, nsuper, super_body, 0)
        plsc.subcore_barrier()

        pltpu.sync_copy(acc.at[pl.ds(row0, rpw)],
                        agg_h.at[pl.ds(pl.multiple_of(c * n_pad + row0, 8), rpw)])

    return k(table, src2, dst128, et2, fi2, zrows)


def _sc_finalize(agg, dst128, x_pad, fi2, ni2, zrows):
    n_pad, H = x_pad.shape
    e_pad = dst128.shape[1] * _CH
    epw = e_pad // _NTILES
    nsub = 8
    ndeg = epw // (nsub * _CH)
    ndrows = n_pad // _CH
    rpw = n_pad // _NTILES
    nrchunks = rpw // _CH
    mesh = plsc.VectorSubcoreMesh(core_axis_name="c", subcore_axis_name="s")

    @functools.partial(
        pl.kernel,
        out_type=jax.ShapeDtypeStruct((_NCORES, n_pad, H), jnp.float32),
        mesh=mesh,
        compiler_params=pltpu.CompilerParams(needs_layout_passes=False),
        scratch_types=[
            pltpu.VMEM_SHARED((ndrows, _CH), jnp.float32),
            pltpu.VMEM((n_pad,), jnp.int32),
            pltpu.VMEM((ndrows, _CH), jnp.float32),
            pltpu.VMEM((ndrows,), jnp.int32),
            pltpu.VMEM((nsub, _CH), jnp.int32),
            pltpu.VMEM((_CH,), jnp.int32),
            pltpu.VMEM((1, _CH), jnp.int32),
            pltpu.VMEM((_CH,), jnp.int32),
            pltpu.VMEM((_CH,), jnp.float32),
            pltpu.VMEM((_CH, H), jnp.float32),
            pltpu.VMEM((_CH, H), jnp.float32),
            pltpu.VMEM((_CH, H), jnp.float32),
            pltpu.SemaphoreType.DMA,
        ],
    )
    def k(agg_h, dst_h, x_h, fi_h, ni_h, z_h,
          out_h,
          deg_sh, fi_v, degv, drow_ids, dst_c, j_c, gix_c, fic, dinv_c,
          rows, xrows, orows, sem):
        c = lax.axis_index("c")
        s = lax.axis_index("s")
        pltpu.sync_copy(fi_h.at[c], fi_v)
        pltpu.sync_copy(z_h.at[pl.ds(0, ndrows)], degv)

        @pl.when(s == 0)
        def _():
            pltpu.sync_copy(z_h.at[pl.ds(0, ndrows)], deg_sh)

        for g in range(ndrows // _LANES):
            drow_ids[pl.ds(g * _LANES, _LANES)] = (
                lax.iota(jnp.int32, _LANES) + g * _LANES)
        ones16 = jnp.ones((_LANES,), jnp.float32)

        def deg_body(kk, carry):
            base128 = pl.multiple_of((s * epw) // _CH + kk * nsub, 8)
            pltpu.sync_copy(dst_h.at[c, pl.ds(base128, nsub)], dst_c)
            for b in range(nsub):
                for g in range(_CH // _LANES):
                    d16 = dst_c[b, pl.ds(g * _LANES, _LANES)]
                    plsc.addupdate_scatter(
                        degv, [lax.shift_right_logical(d16, 7),
                               jnp.bitwise_and(d16, _CH - 1)], ones16)
            return carry

        lax.fori_loop(0, ndeg, deg_body, 0)
        plsc.subcore_barrier()
        pltpu.sync_copy(degv, deg_sh.at[drow_ids], add=True)
        plsc.subcore_barrier()
        pltpu.sync_copy(deg_sh, degv)

        def out_body(kk, carry):
            base = pl.multiple_of(s * rpw + kk * _CH, 8)
            pltpu.sync_copy(ni_h.at[c, pl.ds(base, _CH)], j_c)
            for g in range(_CH // _LANES):
                sl = pl.ds(g * _LANES, _LANES)
                j16 = j_c[sl]
                fic[sl] = plsc.load_gather(fi_v, [j16])
                gix_c[0, sl] = j16 + c * n_pad
                d16 = plsc.load_gather(
                    degv, [lax.shift_right_logical(j16, 7),
                           jnp.bitwise_and(j16, _CH - 1)])
                dinv_c[sl] = 1.0 / jnp.maximum(d16, 1.0)
            pltpu.async_copy(agg_h.at[gix_c.at[0]], rows, sem).wait()
            pltpu.async_copy(x_h.at[fic], xrows, sem).wait()

            def grp_body(g, rc):
                dv16 = dinv_c[pl.ds(pl.multiple_of(g * _LANES, _LANES),
                                    _LANES)]
                for lane in range(_LANES):
                    r = g * _LANES + lane
                    dv = dv16[lane]
                    for cg in range(H // _LANES):
                        slg = pl.ds(cg * _LANES, _LANES)
                        v = rows[r, slg] * dv + xrows[r, slg]
                        orows[r, slg] = jnp.maximum(v, 0.0)
                return rc

            lax.fori_loop(0, _CH // _LANES, grp_body, 0)
            pltpu.sync_copy(orows, out_h.at[c, pl.ds(base, _CH)])
            return carry

        lax.fori_loop(0, nrchunks, out_body, 0)

    return k(agg, dst128, x_pad, fi2, ni2, zrows)


def _fusion_stats(Z0, Z1, W_att, b_att2, q_att2):
    Nn, H = Z0.shape
    I = W_att.shape[1]
    BLK = 2000
    nb = Nn // BLK

    def body(z0_ref, z1_ref, wa_ref, ba_ref, qa_ref, sc_ref, c_ref):
        step = pl.program_id(0)
        a0 = jnp.dot(z0_ref[...], wa_ref[...], preferred_element_type=jnp.float32)
        a1 = jnp.dot(z1_ref[...], wa_ref[...], preferred_element_type=jnp.float32)
        t0 = jnp.tanh(a0 + ba_ref[0][None, :])
        t1 = jnp.tanh(a1 + ba_ref[0][None, :])
        s0 = jnp.sum(t0 * qa_ref[0][None, :])
        s1 = jnp.sum(t1 * qa_ref[0][None, :])
        c01 = lax.dot_general(a0, a1, (((0,), (0,)), ((), ())),
                              preferred_element_type=jnp.float32)
        rr = lax.broadcasted_iota(jnp.int32, (8, 128), 0)
        cc = lax.broadcasted_iota(jnp.int32, (8, 128), 1)
        upd = (jnp.where((rr == 0) & (cc == 0), s0, 0.0)
               + jnp.where((rr == 0) & (cc == 1), s1, 0.0))

        @pl.when(step == 0)
        def _():
            sc_ref[...] = jnp.zeros_like(sc_ref)
            c_ref[...] = jnp.zeros_like(c_ref)

        sc_ref[...] += upd
        c_ref[...] += c01

    return pl.pallas_call(
        body,
        grid=(nb,),
        in_specs=[
            pl.BlockSpec((BLK, H), lambda b: (b, 0)),
            pl.BlockSpec((BLK, H), lambda b: (b, 0)),
            pl.BlockSpec((H, I), lambda b: (0, 0)),
            pl.BlockSpec((1, I), lambda b: (0, 0)),
            pl.BlockSpec((1, I), lambda b: (0, 0)),
        ],
        out_specs=[
            pl.BlockSpec((8, 128), lambda b: (0, 0)),
            pl.BlockSpec((I, I), lambda b: (0, 0)),
        ],
        out_shape=[
            jax.ShapeDtypeStruct((8, 128), jnp.float32),
            jax.ShapeDtypeStruct((I, I), jnp.float32),
        ],
    )(Z0, Z1, W_att, b_att2, q_att2)


def _fusion_out(Z0, Z1, scores, c01, W_dec):
    Nn, H = Z0.shape
    I = c01.shape[0]
    BLK = 2000
    nb = Nn // BLK

    def body(z0_ref, z1_ref, sc_ref, c_ref, wd_ref, f_ref, l_ref, acc_ref):
        step = pl.program_id(0)
        scv = sc_ref[...]
        rr = lax.broadcasted_iota(jnp.int32, (8, 128), 0)
        cc = lax.broadcasted_iota(jnp.int32, (8, 128), 1)
        s0 = jnp.sum(jnp.where((rr == 0) & (cc == 0), scv, 0.0)) / Nn
        s1 = jnp.sum(jnp.where((rr == 0) & (cc == 1), scv, 0.0)) / Nn
        m = jnp.maximum(s0, s1)
        e0 = jnp.exp(s0 - m)
        e1 = jnp.exp(s1 - m)
        a0 = e0 / (e0 + e1)
        a1 = e1 / (e0 + e1)
        z0 = z0_ref[...]
        z1 = z1_ref[...]
        fused = a0 * z0 + a1 * z1
        f_ref[...] = fused
        r0 = jnp.dot(fused, wd_ref[0], preferred_element_type=jnp.float32)
        r1 = jnp.dot(fused, wd_ref[1], preferred_element_type=jnp.float32)
        se = jnp.sum((r0 - z0) ** 2) + jnp.sum((r1 - z1) ** 2)

        @pl.when(step == 0)
        def _():
            acc_ref[0] = 0.0

        acc_ref[0] += se

        @pl.when(step == nb - 1)
        def _():
            cmat = c_ref[...] / Nn
            ortho = jnp.sum(cmat * cmat)
            l_ref[...] = jnp.full((1, 1), acc_ref[0] / (2.0 * Nn * H) + ortho,
                                  jnp.float32)

    return pl.pallas_call(
        body,
        grid=(nb,),
        in_specs=[
            pl.BlockSpec((BLK, H), lambda b: (b, 0)),
            pl.BlockSpec((BLK, H), lambda b: (b, 0)),
            pl.BlockSpec((8, 128), lambda b: (0, 0)),
            pl.BlockSpec((I, I), lambda b: (0, 0)),
            pl.BlockSpec((2, H, H), lambda b: (0, 0, 0)),
        ],
        out_specs=[
            pl.BlockSpec((BLK, H), lambda b: (b, 0)),
            pl.BlockSpec((1, 1), lambda b: (0, 0)),
        ],
        out_shape=[
            jax.ShapeDtypeStruct((Nn, H), jnp.float32),
            jax.ShapeDtypeStruct((1, 1), jnp.float32),
        ],
        scratch_shapes=[pltpu.SMEM((1,), jnp.float32)],
    )(Z0, Z1, scores, c01, W_dec)


def kernel(transformed_features, edge_index_0, edge_type_0, new_index_0,
           feature_index_0, edge_index_1, edge_type_1, new_index_1,
           feature_index_1, W_rel, b_rel, W_att, b_att, q_att, W_dec):
    N, H = transformed_features.shape
    E = edge_type_0.shape[0]
    R = W_rel.shape[0]
    grain = _NTILES * _CH
    n_pad = -(-N // grain) * grain
    egrain = _NTILES * 1024
    e_pad = -(-E // egrain) * egrain

    x_pad = jnp.pad(transformed_features, ((0, n_pad - N), (0, 0)))
    src2 = jnp.pad(jnp.stack([edge_index_0[0], edge_index_1[0]]),
                   ((0, 0), (0, e_pad - E)))
    dst128 = jnp.pad(jnp.stack([edge_index_0[1], edge_index_1[1]]),
                     ((0, 0), (0, e_pad - E)),
                     constant_values=N).reshape(_NCORES, e_pad // _CH, _CH)
    et2 = jnp.pad(jnp.stack([edge_type_0, edge_type_1]),
                  ((0, 0), (0, e_pad - E)))
    fi2 = jnp.pad(jnp.stack([feature_index_0, feature_index_1]),
                  ((0, 0), (0, n_pad - N)))
    ni2 = jnp.pad(jnp.stack([new_index_0, new_index_1]),
                  ((0, 0), (0, n_pad - N)))
    zrows = jnp.zeros((n_pad // _NTILES, H), jnp.float32)

    table = _build_table(x_pad, W_rel, b_rel.reshape(R, 1, H))
    agg = _sc_edges(table, src2, dst128, et2, fi2, zrows)
    Z = _sc_finalize(agg, dst128, x_pad, fi2, ni2, zrows)
    Z0 = Z[0, :N]
    Z1 = Z[1, :N]
    scores, c01 = _fusion_stats(Z0, Z1, W_att, b_att.reshape(1, -1),
                                q_att.reshape(1, -1))
    fused, loss = _fusion_out(Z0, Z1, scores, c01, W_dec)
    return fused, loss[0, 0]


# speedup vs baseline: 1.0282x; 1.0282x over previous
import functools

import jax
import jax.numpy as jnp
from jax import lax
from jax.experimental import pallas as pl
from jax.experimental.pallas import tpu as pltpu
from jax.experimental.pallas import tpu_sc as plsc

_LANES = 16
_NTILES = 16
_NCORES = 2
_CH = 128


def _build_table(x_pad, W_rel, b_rel3):
    n_pad, H = x_pad.shape
    R = W_rel.shape[0]
    BLK = 1024
    nb = n_pad // BLK

    def body(x_ref, w_ref, b_ref, o_ref):
        mm = jnp.dot(x_ref[...], w_ref[0], preferred_element_type=jnp.float32)
        o_ref[0] = mm + b_ref[0, 0][None, :]

    out = pl.pallas_call(
        body,
        grid=(R, nb),
        in_specs=[
            pl.BlockSpec((BLK, H), lambda r, b: (b, 0)),
            pl.BlockSpec((1, H, H), lambda r, b: (r, 0, 0)),
            pl.BlockSpec((1, 1, H), lambda r, b: (r, 0, 0)),
        ],
        out_specs=pl.BlockSpec((1, BLK, H), lambda r, b: (r, b, 0)),
        out_shape=jax.ShapeDtypeStruct((R, n_pad, H), jnp.float32),
    )(x_pad, W_rel, b_rel3)
    return out.reshape(R * n_pad, H)


def _sc_indices(src2, et2, fi2):
    n_pad = fi2.shape[1]
    e_pad = src2.shape[1]
    epw = e_pad // _NTILES
    SUP = 1024
    nsub = SUP // _CH
    nsuper = epw // SUP
    mesh = plsc.VectorSubcoreMesh(core_axis_name="c", subcore_axis_name="s")

    @functools.partial(
        pl.kernel,
        out_type=jax.ShapeDtypeStruct((_NCORES, e_pad // _CH, _CH), jnp.int32),
        mesh=mesh,
        compiler_params=pltpu.CompilerParams(needs_layout_passes=False),
        scratch_types=[
            pltpu.VMEM((n_pad,), jnp.int32),
            pltpu.VMEM((SUP,), jnp.int32),
            pltpu.VMEM((SUP,), jnp.int32),
            pltpu.VMEM((nsub, _CH), jnp.int32),
        ],
    )
    def k(src_h, et_h, fi_h, gix_h, fi_v, src_c, et_c, gix_c):
        c = lax.axis_index("c")
        s = lax.axis_index("s")
        pltpu.sync_copy(fi_h.at[c], fi_v)

        def super_body(kk, carry):
            base = pl.multiple_of(s * epw + kk * SUP, 8)
            base128 = pl.multiple_of((s * epw + kk * SUP) // _CH, 8)
            pltpu.sync_copy(src_h.at[c, pl.ds(base, SUP)], src_c)
            pltpu.sync_copy(et_h.at[c, pl.ds(base, SUP)], et_c)
            for b in range(nsub):
                for g in range(_CH // _LANES):
                    sl = pl.ds(g * _LANES, _LANES)
                    sle = pl.ds(b * _CH + g * _LANES, _LANES)
                    f16 = plsc.load_gather(fi_v, [src_c[sle]])
                    gix_c[b, sl] = et_c[sle] * n_pad + f16
            pltpu.sync_copy(gix_c, gix_h.at[c, pl.ds(base128, nsub)])
            return carry

        lax.fori_loop(0, nsuper, super_body, 0)

    return k(src2, et2, fi2)


def _sc_edges(table, gix2, dst128, fi2, zrows):
    n_pad = fi2.shape[1]
    H = table.shape[1]
    e_pad = gix2.shape[1] * _CH
    epw = e_pad // _NTILES
    SUP = 2048
    nsub = SUP // _CH
    nsuper = epw // SUP
    rpw = n_pad // _NTILES
    mesh = plsc.VectorSubcoreMesh(core_axis_name="c", subcore_axis_name="s")

    @functools.partial(
        pl.kernel,
        out_type=jax.ShapeDtypeStruct((_NCORES * n_pad, H), jnp.float32),
        mesh=mesh,
        compiler_params=pltpu.CompilerParams(needs_layout_passes=False),
        scratch_types=[
            pltpu.VMEM_SHARED((n_pad, H), jnp.float32),
            pltpu.VMEM((nsub, _CH), jnp.int32),
            pltpu.VMEM((nsub, _CH), jnp.int32),
            pltpu.VMEM((_CH, H), jnp.float32),
            pltpu.VMEM((_CH, H), jnp.float32),
            pltpu.SemaphoreType.DMA,
            pltpu.SemaphoreType.DMA,
            pltpu.SemaphoreType.DMA,
            pltpu.SemaphoreType.DMA,
        ],
    )
    def k(table_h, gix_h, dst_h, fi_h, z_h,
          agg_h,
          acc, dst_c, gix_c, rows0, rows1,
          gsem0, gsem1, ssem0, ssem1):
        c = lax.axis_index("c")
        s = lax.axis_index("s")
        row0 = pl.multiple_of(s * rpw, 8)
        pltpu.sync_copy(z_h, acc.at[pl.ds(row0, rpw)])
        plsc.subcore_barrier()

        rbufs = (rows0, rows1)
        gsems = (gsem0, gsem1)
        ssems = (ssem0, ssem1)

        def super_body(kk, carry):
            base128 = pl.multiple_of((s * epw + kk * SUP) // _CH, 8)
            pltpu.sync_copy(gix_h.at[c, pl.ds(base128, nsub)], gix_c)
            pltpu.sync_copy(dst_h.at[c, pl.ds(base128, nsub)], dst_c)
            gat = [None, None]
            scat = [None, None]
            gat[0] = pltpu.async_copy(table_h.at[gix_c.at[0]], rbufs[0],
                                      gsems[0])
            for b in range(nsub):
                i = b & 1
                if b + 1 < nsub:
                    j = (b + 1) & 1
                    if scat[j] is not None:
                        scat[j].wait()
                    gat[j] = pltpu.async_copy(
                        table_h.at[gix_c.at[b + 1]], rbufs[j], gsems[j])
                gat[i].wait()
                scat[i] = pltpu.async_copy(rbufs[i], acc.at[dst_c.at[b]],
                                           ssems[i], add=True)
            scat[0].wait()
            scat[1].wait()
            return carry

        lax.fori_loop(0, nsuper, super_body, 0)
        plsc.subcore_barrier()

        pltpu.sync_copy(acc.at[pl.ds(row0, rpw)],
                        agg_h.at[pl.ds(pl.multiple_of(c * n_pad + row0, 8), rpw)])

    return k(table, gix2, dst128, fi2, zrows)


def _sc_finalize(agg, dst128, x_pad, fi2, ni2, zrows):
    n_pad, H = x_pad.shape
    e_pad = dst128.shape[1] * _CH
    epw = e_pad // _NTILES
    nsub = 8
    ndeg = epw // (nsub * _CH)
    ndrows = n_pad // _CH
    rpw = n_pad // _NTILES
    nrchunks = rpw // _CH
    mesh = plsc.VectorSubcoreMesh(core_axis_name="c", subcore_axis_name="s")

    @functools.partial(
        pl.kernel,
        out_type=jax.ShapeDtypeStruct((_NCORES, n_pad, H), jnp.float32),
        mesh=mesh,
        compiler_params=pltpu.CompilerParams(needs_layout_passes=False),
        scratch_types=[
            pltpu.VMEM_SHARED((ndrows, _CH), jnp.float32),
            pltpu.VMEM((n_pad,), jnp.int32),
            pltpu.VMEM((ndrows, _CH), jnp.float32),
            pltpu.VMEM((ndrows,), jnp.int32),
            pltpu.VMEM((nsub, _CH), jnp.int32),
            pltpu.VMEM((_CH,), jnp.int32),
            pltpu.VMEM((1, _CH), jnp.int32),
            pltpu.VMEM((_CH,), jnp.int32),
            pltpu.VMEM((_CH,), jnp.float32),
            pltpu.VMEM((_CH, H), jnp.float32),
            pltpu.VMEM((_CH, H), jnp.float32),
            pltpu.VMEM((_CH, H), jnp.float32),
            pltpu.SemaphoreType.DMA,
        ],
    )
    def k(agg_h, dst_h, x_h, fi_h, ni_h, z_h,
          out_h,
          deg_sh, fi_v, degv, drow_ids, dst_c, j_c, gix_c, fic, dinv_c,
          rows, xrows, orows, sem):
        c = lax.axis_index("c")
        s = lax.axis_index("s")
        pltpu.sync_copy(fi_h.at[c], fi_v)
        pltpu.sync_copy(z_h.at[pl.ds(0, ndrows)], degv)

        @pl.when(s == 0)
        def _():
            pltpu.sync_copy(z_h.at[pl.ds(0, ndrows)], deg_sh)

        for g in range(ndrows // _LANES):
            drow_ids[pl.ds(g * _LANES, _LANES)] = (
                lax.iota(jnp.int32, _LANES) + g * _LANES)
        ones16 = jnp.ones((_LANES,), jnp.float32)

        def deg_body(kk, carry):
            base128 = pl.multiple_of((s * epw) // _CH + kk * nsub, 8)
            pltpu.sync_copy(dst_h.at[c, pl.ds(base128, nsub)], dst_c)
            for b in range(nsub):
                for g in range(_CH // _LANES):
                    d16 = dst_c[b, pl.ds(g * _LANES, _LANES)]
                    plsc.addupdate_scatter(
                        degv, [lax.shift_right_logical(d16, 7),
                               jnp.bitwise_and(d16, _CH - 1)], ones16)
            return carry

        lax.fori_loop(0, ndeg, deg_body, 0)
        plsc.subcore_barrier()
        pltpu.sync_copy(degv, deg_sh.at[drow_ids], add=True)
        plsc.subcore_barrier()
        pltpu.sync_copy(deg_sh, degv)

        def out_body(kk, carry):
            base = pl.multiple_of(s * rpw + kk * _CH, 8)
            pltpu.sync_copy(ni_h.at[c, pl.ds(base, _CH)], j_c)
            for g in range(_CH // _LANES):
                sl = pl.ds(g * _LANES, _LANES)
                j16 = j_c[sl]
                fic[sl] = plsc.load_gather(fi_v, [j16])
                gix_c[0, sl] = j16 + c * n_pad
                d16 = plsc.load_gather(
                    degv, [lax.shift_right_logical(j16, 7),
                           jnp.bitwise_and(j16, _CH - 1)])
                dinv_c[sl] = 1.0 / jnp.maximum(d16, 1.0)
            pltpu.async_copy(agg_h.at[gix_c.at[0]], rows, sem).wait()
            pltpu.async_copy(x_h.at[fic], xrows, sem).wait()

            def grp_body(g, rc):
                dv16 = dinv_c[pl.ds(pl.multiple_of(g * _LANES, _LANES),
                                    _LANES)]
                for lane in range(_LANES):
                    r = g * _LANES + lane
                    dv = dv16[lane]
                    for cg in range(H // _LANES):
                        slg = pl.ds(cg * _LANES, _LANES)
                        v = rows[r, slg] * dv + xrows[r, slg]
                        orows[r, slg] = jnp.maximum(v, 0.0)
                return rc

            lax.fori_loop(0, _CH // _LANES, grp_body, 0)
            pltpu.sync_copy(orows, out_h.at[c, pl.ds(base, _CH)])
            return carry

        lax.fori_loop(0, nrchunks, out_body, 0)

    return k(agg, dst128, x_pad, fi2, ni2, zrows)


def _fusion_stats(Z0, Z1, W_att, b_att2, q_att2):
    Nn, H = Z0.shape
    I = W_att.shape[1]
    BLK = 2000
    nb = Nn // BLK

    def body(z0_ref, z1_ref, wa_ref, ba_ref, qa_ref, sc_ref, c_ref):
        step = pl.program_id(0)
        a0 = jnp.dot(z0_ref[...], wa_ref[...], preferred_element_type=jnp.float32)
        a1 = jnp.dot(z1_ref[...], wa_ref[...], preferred_element_type=jnp.float32)
        t0 = jnp.tanh(a0 + ba_ref[0][None, :])
        t1 = jnp.tanh(a1 + ba_ref[0][None, :])
        s0 = jnp.sum(t0 * qa_ref[0][None, :])
        s1 = jnp.sum(t1 * qa_ref[0][None, :])
        c01 = lax.dot_general(a0, a1, (((0,), (0,)), ((), ())),
                              preferred_element_type=jnp.float32)
        rr = lax.broadcasted_iota(jnp.int32, (8, 128), 0)
        cc = lax.broadcasted_iota(jnp.int32, (8, 128), 1)
        upd = (jnp.where((rr == 0) & (cc == 0), s0, 0.0)
               + jnp.where((rr == 0) & (cc == 1), s1, 0.0))

        @pl.when(step == 0)
        def _():
            sc_ref[...] = jnp.zeros_like(sc_ref)
            c_ref[...] = jnp.zeros_like(c_ref)

        sc_ref[...] += upd
        c_ref[...] += c01

    return pl.pallas_call(
        body,
        grid=(nb,),
        in_specs=[
            pl.BlockSpec((BLK, H), lambda b: (b, 0)),
            pl.BlockSpec((BLK, H), lambda b: (b, 0)),
            pl.BlockSpec((H, I), lambda b: (0, 0)),
            pl.BlockSpec((1, I), lambda b: (0, 0)),
            pl.BlockSpec((1, I), lambda b: (0, 0)),
        ],
        out_specs=[
            pl.BlockSpec((8, 128), lambda b: (0, 0)),
            pl.BlockSpec((I, I), lambda b: (0, 0)),
        ],
        out_shape=[
            jax.ShapeDtypeStruct((8, 128), jnp.float32),
            jax.ShapeDtypeStruct((I, I), jnp.float32),
        ],
    )(Z0, Z1, W_att, b_att2, q_att2)


def _fusion_out(Z0, Z1, scores, c01, W_dec):
    Nn, H = Z0.shape
    I = c01.shape[0]
    BLK = 2000
    nb = Nn // BLK

    def body(z0_ref, z1_ref, sc_ref, c_ref, wd_ref, f_ref, l_ref, acc_ref):
        step = pl.program_id(0)
        scv = sc_ref[...]
        rr = lax.broadcasted_iota(jnp.int32, (8, 128), 0)
        cc = lax.broadcasted_iota(jnp.int32, (8, 128), 1)
        s0 = jnp.sum(jnp.where((rr == 0) & (cc == 0), scv, 0.0)) / Nn
        s1 = jnp.sum(jnp.where((rr == 0) & (cc == 1), scv, 0.0)) / Nn
        m = jnp.maximum(s0, s1)
        e0 = jnp.exp(s0 - m)
        e1 = jnp.exp(s1 - m)
        a0 = e0 / (e0 + e1)
        a1 = e1 / (e0 + e1)
        z0 = z0_ref[...]
        z1 = z1_ref[...]
        fused = a0 * z0 + a1 * z1
        f_ref[...] = fused
        r0 = jnp.dot(fused, wd_ref[0], preferred_element_type=jnp.float32)
        r1 = jnp.dot(fused, wd_ref[1], preferred_element_type=jnp.float32)
        se = jnp.sum((r0 - z0) ** 2) + jnp.sum((r1 - z1) ** 2)

        @pl.when(step == 0)
        def _():
            acc_ref[0] = 0.0

        acc_ref[0] += se

        @pl.when(step == nb - 1)
        def _():
            cmat = c_ref[...] / Nn
            ortho = jnp.sum(cmat * cmat)
            l_ref[...] = jnp.full((1, 1), acc_ref[0] / (2.0 * Nn * H) + ortho,
                                  jnp.float32)

    return pl.pallas_call(
        body,
        grid=(nb,),
        in_specs=[
            pl.BlockSpec((BLK, H), lambda b: (b, 0)),
            pl.BlockSpec((BLK, H), lambda b: (b, 0)),
            pl.BlockSpec((8, 128), lambda b: (0, 0)),
            pl.BlockSpec((I, I), lambda b: (0, 0)),
            pl.BlockSpec((2, H, H), lambda b: (0, 0, 0)),
        ],
        out_specs=[
            pl.BlockSpec((BLK, H), lambda b: (b, 0)),
            pl.BlockSpec((1, 1), lambda b: (0, 0)),
        ],
        out_shape=[
            jax.ShapeDtypeStruct((Nn, H), jnp.float32),
            jax.ShapeDtypeStruct((1, 1), jnp.float32),
        ],
        scratch_shapes=[pltpu.SMEM((1,), jnp.float32)],
    )(Z0, Z1, scores, c01, W_dec)


def kernel(transformed_features, edge_index_0, edge_type_0, new_index_0,
           feature_index_0, edge_index_1, edge_type_1, new_index_1,
           feature_index_1, W_rel, b_rel, W_att, b_att, q_att, W_dec):
    N, H = transformed_features.shape
    E = edge_type_0.shape[0]
    R = W_rel.shape[0]
    grain = _NTILES * _CH
    n_pad = -(-N // grain) * grain
    egrain = _NTILES * 1024
    e_pad = -(-E // egrain) * egrain

    x_pad = jnp.pad(transformed_features, ((0, n_pad - N), (0, 0)))
    src2 = jnp.pad(jnp.stack([edge_index_0[0], edge_index_1[0]]),
                   ((0, 0), (0, e_pad - E)))
    dst128 = jnp.pad(jnp.stack([edge_index_0[1], edge_index_1[1]]),
                     ((0, 0), (0, e_pad - E)),
                     constant_values=N).reshape(_NCORES, e_pad // _CH, _CH)
    et2 = jnp.pad(jnp.stack([edge_type_0, edge_type_1]),
                  ((0, 0), (0, e_pad - E)))
    fi2 = jnp.pad(jnp.stack([feature_index_0, feature_index_1]),
                  ((0, 0), (0, n_pad - N)))
    ni2 = jnp.pad(jnp.stack([new_index_0, new_index_1]),
                  ((0, 0), (0, n_pad - N)))
    zrows = jnp.zeros((n_pad // _NTILES, H), jnp.float32)

    table = _build_table(x_pad, W_rel, b_rel.reshape(R, 1, H))
    gix2 = _sc_indices(src2, et2, fi2)
    agg = _sc_edges(table, gix2, dst128, fi2, zrows)
    Z = _sc_finalize(agg, dst128, x_pad, fi2, ni2, zrows)
    Z0 = Z[0, :N]
    Z1 = Z[1, :N]
    scores, c01 = _fusion_stats(Z0, Z1, W_att, b_att.reshape(1, -1),
                                q_att.reshape(1, -1))
    fused, loss = _fusion_out(Z0, Z1, scores, c01, W_dec)
    return fused, loss[0, 0]


# speedup vs baseline: 1.0587x; 1.0297x over previous
import functools

import jax
import jax.numpy as jnp
from jax import lax
from jax.experimental import pallas as pl
from jax.experimental.pallas import tpu as pltpu
from jax.experimental.pallas import tpu_sc as plsc

_LANES = 16
_NTILES = 16
_NCORES = 2
_CH = 128


def _build_table(x_pad, W_rel, b_rel3):
    n_pad, H = x_pad.shape
    R = W_rel.shape[0]
    BLK = 1024
    nb = n_pad // BLK

    def body(x_ref, w_ref, b_ref, o_ref):
        mm = jnp.dot(x_ref[...], w_ref[0], preferred_element_type=jnp.float32)
        o_ref[0] = mm + b_ref[0, 0][None, :]

    out = pl.pallas_call(
        body,
        grid=(R, nb),
        in_specs=[
            pl.BlockSpec((BLK, H), lambda r, b: (b, 0)),
            pl.BlockSpec((1, H, H), lambda r, b: (r, 0, 0)),
            pl.BlockSpec((1, 1, H), lambda r, b: (r, 0, 0)),
        ],
        out_specs=pl.BlockSpec((1, BLK, H), lambda r, b: (r, b, 0)),
        out_shape=jax.ShapeDtypeStruct((R, n_pad, H), jnp.float32),
    )(x_pad, W_rel, b_rel3)
    return out.reshape(R * n_pad, H)


def _sc_edges(table, src2, dst2, et2, fi2, zrows):
    n_pad = fi2.shape[1]
    H = table.shape[1]
    e_pad = src2.shape[1]
    epw = e_pad // _NTILES
    nchunks = epw // _CH
    rpw = n_pad // _NTILES
    ndrows = n_pad // _CH
    mesh = plsc.VectorSubcoreMesh(core_axis_name="c", subcore_axis_name="s")

    @functools.partial(
        pl.kernel,
        out_type=(
            jax.ShapeDtypeStruct((_NCORES * n_pad, H), jnp.float32),
            jax.ShapeDtypeStruct((_NCORES, ndrows, _CH), jnp.float32),
        ),
        mesh=mesh,
        compiler_params=pltpu.CompilerParams(needs_layout_passes=False),
        scratch_types=[
            pltpu.VMEM_SHARED((n_pad, H), jnp.float32),
            pltpu.VMEM_SHARED((ndrows, _CH), jnp.float32),
            pltpu.VMEM((n_pad,), jnp.int32),
            pltpu.VMEM((_CH,), jnp.int32),
            pltpu.VMEM((_CH,), jnp.int32),
            pltpu.VMEM((1, _CH), jnp.int32),
            pltpu.VMEM((1, _CH), jnp.int32),
            pltpu.VMEM((_CH, H), jnp.float32),
            pltpu.VMEM((ndrows, _CH), jnp.float32),
            pltpu.VMEM((ndrows,), jnp.int32),
            pltpu.SemaphoreType.DMA,
        ],
    )
    def k(table_h, src_h, dst_h, et_h, fi_h, z_h,
          agg_h, deg_h,
          acc, deg_sh, fi_v, src_c, et_c, dst_c, gix_c, rows,
          degp, drow_ids, sem):
        c = lax.axis_index("c")
        s = lax.axis_index("s")
        pltpu.sync_copy(fi_h.at[c], fi_v)
        row0 = pl.multiple_of(s * rpw, 8)
        pltpu.sync_copy(z_h, acc.at[pl.ds(row0, rpw)])
        pltpu.sync_copy(z_h.at[pl.ds(0, ndrows)], degp)

        @pl.when(s == 0)
        def _():
            pltpu.sync_copy(z_h.at[pl.ds(0, ndrows)], deg_sh)

        for g in range(ndrows // _LANES):
            drow_ids[pl.ds(g * _LANES, _LANES)] = (
                lax.iota(jnp.int32, _LANES) + g * _LANES)
        plsc.subcore_barrier()

        def chunk_body(kk, carry):
            base = pl.multiple_of(s * epw + kk * _CH, 8)
            pltpu.sync_copy(src_h.at[c, pl.ds(base, _CH)], src_c)
            pltpu.sync_copy(et_h.at[c, pl.ds(base, _CH)], et_c)
            pltpu.sync_copy(dst_h.at[c, pl.ds(base, _CH)], dst_c.at[0])
            ones16 = jnp.ones((_LANES,), jnp.float32)
            for g in range(_CH // _LANES):
                sl = pl.ds(g * _LANES, _LANES)
                f16 = plsc.load_gather(fi_v, [src_c[sl]])
                gix_c[0, sl] = et_c[sl] * n_pad + f16
                d16 = dst_c[0, sl]
                plsc.addupdate_scatter(
                    degp, [lax.shift_right_logical(d16, 7),
                           jnp.bitwise_and(d16, _CH - 1)], ones16)
            pltpu.async_copy(table_h.at[gix_c.at[0]], rows, sem).wait()
            pltpu.sync_copy(rows, acc.at[dst_c.at[0]], add=True)
            return carry

        lax.fori_loop(0, nchunks, chunk_body, 0)
        pltpu.sync_copy(degp, deg_sh.at[drow_ids], add=True)
        plsc.subcore_barrier()

        pltpu.sync_copy(acc.at[pl.ds(row0, rpw)],
                        agg_h.at[pl.ds(pl.multiple_of(c * n_pad + row0, 8), rpw)])

        @pl.when(s == 0)
        def _():
            pltpu.sync_copy(deg_sh, deg_h.at[c])

    return k(table, src2, dst2, et2, fi2, zrows)


def _sc_finalize(agg, deg, x_pad, fi2, ni2):
    n_pad, H = x_pad.shape
    ndrows = n_pad // _CH
    rpw = n_pad // _NTILES
    nrchunks = rpw // _CH
    mesh = plsc.VectorSubcoreMesh(core_axis_name="c", subcore_axis_name="s")

    @functools.partial(
        pl.kernel,
        out_type=jax.ShapeDtypeStruct((_NCORES, n_pad, H), jnp.float32),
        mesh=mesh,
        compiler_params=pltpu.CompilerParams(needs_layout_passes=False),
        scratch_types=[
            pltpu.VMEM((n_pad,), jnp.int32),
            pltpu.VMEM((ndrows, _CH), jnp.float32),
            pltpu.VMEM((_CH,), jnp.int32),
            pltpu.VMEM((1, _CH), jnp.int32),
            pltpu.VMEM((_CH,), jnp.int32),
            pltpu.VMEM((_CH,), jnp.float32),
            pltpu.VMEM((_CH, H), jnp.float32),
            pltpu.VMEM((_CH, H), jnp.float32),
            pltpu.VMEM((_CH, H), jnp.float32),
            pltpu.SemaphoreType.DMA,
        ],
    )
    def k(agg_h, deg_h, x_h, fi_h, ni_h,
          out_h,
          fi_v, degv, j_c, gix_c, fic, dinv_c, rows, xrows, orows, sem):
        c = lax.axis_index("c")
        s = lax.axis_index("s")
        pltpu.sync_copy(fi_h.at[c], fi_v)
        pltpu.sync_copy(deg_h.at[c], degv)

        def out_body(kk, carry):
            base = pl.multiple_of(s * rpw + kk * _CH, 8)
            pltpu.sync_copy(ni_h.at[c, pl.ds(base, _CH)], j_c)
            for g in range(_CH // _LANES):
                sl = pl.ds(g * _LANES, _LANES)
                j16 = j_c[sl]
                fic[sl] = plsc.load_gather(fi_v, [j16])
                gix_c[0, sl] = j16 + c * n_pad
                d16 = plsc.load_gather(
                    degv, [lax.shift_right_logical(j16, 7),
                           jnp.bitwise_and(j16, _CH - 1)])
                dinv_c[sl] = 1.0 / jnp.maximum(d16, 1.0)
            pltpu.async_copy(agg_h.at[gix_c.at[0]], rows, sem).wait()
            pltpu.async_copy(x_h.at[fic], xrows, sem).wait()

            def grp_body(g, rc):
                dv16 = dinv_c[pl.ds(pl.multiple_of(g * _LANES, _LANES),
                                    _LANES)]
                for lane in range(_LANES):
                    r = g * _LANES + lane
                    dv = dv16[lane]
                    for cg in range(H // _LANES):
                        slg = pl.ds(cg * _LANES, _LANES)
                        v = rows[r, slg] * dv + xrows[r, slg]
                        orows[r, slg] = jnp.maximum(v, 0.0)
                return rc

            lax.fori_loop(0, _CH // _LANES, grp_body, 0)
            pltpu.sync_copy(orows, out_h.at[c, pl.ds(base, _CH)])
            return carry

        lax.fori_loop(0, nrchunks, out_body, 0)

    return k(agg, deg, x_pad, fi2, ni2)


def _fusion_stats(Z0, Z1, W_att, b_att2, q_att2):
    Nn, H = Z0.shape
    I = W_att.shape[1]
    BLK = 2000
    nb = Nn // BLK

    def body(z0_ref, z1_ref, wa_ref, ba_ref, qa_ref, sc_ref, c_ref):
        step = pl.program_id(0)
        a0 = jnp.dot(z0_ref[...], wa_ref[...], preferred_element_type=jnp.float32)
        a1 = jnp.dot(z1_ref[...], wa_ref[...], preferred_element_type=jnp.float32)
        t0 = jnp.tanh(a0 + ba_ref[0][None, :])
        t1 = jnp.tanh(a1 + ba_ref[0][None, :])
        s0 = jnp.sum(t0 * qa_ref[0][None, :])
        s1 = jnp.sum(t1 * qa_ref[0][None, :])
        c01 = lax.dot_general(a0, a1, (((0,), (0,)), ((), ())),
                              preferred_element_type=jnp.float32)
        rr = lax.broadcasted_iota(jnp.int32, (8, 128), 0)
        cc = lax.broadcasted_iota(jnp.int32, (8, 128), 1)
        upd = (jnp.where((rr == 0) & (cc == 0), s0, 0.0)
               + jnp.where((rr == 0) & (cc == 1), s1, 0.0))

        @pl.when(step == 0)
        def _():
            sc_ref[...] = jnp.zeros_like(sc_ref)
            c_ref[...] = jnp.zeros_like(c_ref)

        sc_ref[...] += upd
        c_ref[...] += c01

    return pl.pallas_call(
        body,
        grid=(nb,),
        in_specs=[
            pl.BlockSpec((BLK, H), lambda b: (b, 0)),
            pl.BlockSpec((BLK, H), lambda b: (b, 0)),
            pl.BlockSpec((H, I), lambda b: (0, 0)),
            pl.BlockSpec((1, I), lambda b: (0, 0)),
            pl.BlockSpec((1, I), lambda b: (0, 0)),
        ],
        out_specs=[
            pl.BlockSpec((8, 128), lambda b: (0, 0)),
            pl.BlockSpec((I, I), lambda b: (0, 0)),
        ],
        out_shape=[
            jax.ShapeDtypeStruct((8, 128), jnp.float32),
            jax.ShapeDtypeStruct((I, I), jnp.float32),
        ],
    )(Z0, Z1, W_att, b_att2, q_att2)


def _fusion_out(Z0, Z1, scores, c01, W_dec):
    Nn, H = Z0.shape
    I = c01.shape[0]
    BLK = 2000
    nb = Nn // BLK

    def body(z0_ref, z1_ref, sc_ref, c_ref, wd_ref, f_ref, l_ref, acc_ref):
        step = pl.program_id(0)
        scv = sc_ref[...]
        rr = lax.broadcasted_iota(jnp.int32, (8, 128), 0)
        cc = lax.broadcasted_iota(jnp.int32, (8, 128), 1)
        s0 = jnp.sum(jnp.where((rr == 0) & (cc == 0), scv, 0.0)) / Nn
        s1 = jnp.sum(jnp.where((rr == 0) & (cc == 1), scv, 0.0)) / Nn
        m = jnp.maximum(s0, s1)
        e0 = jnp.exp(s0 - m)
        e1 = jnp.exp(s1 - m)
        a0 = e0 / (e0 + e1)
        a1 = e1 / (e0 + e1)
        z0 = z0_ref[...]
        z1 = z1_ref[...]
        fused = a0 * z0 + a1 * z1
        f_ref[...] = fused
        r0 = jnp.dot(fused, wd_ref[0], preferred_element_type=jnp.float32)
        r1 = jnp.dot(fused, wd_ref[1], preferred_element_type=jnp.float32)
        se = jnp.sum((r0 - z0) ** 2) + jnp.sum((r1 - z1) ** 2)

        @pl.when(step == 0)
        def _():
            acc_ref[0] = 0.0

        acc_ref[0] += se

        @pl.when(step == nb - 1)
        def _():
            cmat = c_ref[...] / Nn
            ortho = jnp.sum(cmat * cmat)
            l_ref[...] = jnp.full((1, 1), acc_ref[0] / (2.0 * Nn * H) + ortho,
                                  jnp.float32)

    return pl.pallas_call(
        body,
        grid=(nb,),
        in_specs=[
            pl.BlockSpec((BLK, H), lambda b: (b, 0)),
            pl.BlockSpec((BLK, H), lambda b: (b, 0)),
            pl.BlockSpec((8, 128), lambda b: (0, 0)),
            pl.BlockSpec((I, I), lambda b: (0, 0)),
            pl.BlockSpec((2, H, H), lambda b: (0, 0, 0)),
        ],
        out_specs=[
            pl.BlockSpec((BLK, H), lambda b: (b, 0)),
            pl.BlockSpec((1, 1), lambda b: (0, 0)),
        ],
        out_shape=[
            jax.ShapeDtypeStruct((Nn, H), jnp.float32),
            jax.ShapeDtypeStruct((1, 1), jnp.float32),
        ],
        scratch_shapes=[pltpu.SMEM((1,), jnp.float32)],
    )(Z0, Z1, scores, c01, W_dec)


def kernel(transformed_features, edge_index_0, edge_type_0, new_index_0,
           feature_index_0, edge_index_1, edge_type_1, new_index_1,
           feature_index_1, W_rel, b_rel, W_att, b_att, q_att, W_dec):
    N, H = transformed_features.shape
    E = edge_type_0.shape[0]
    R = W_rel.shape[0]
    grain = _NTILES * _CH
    n_pad = -(-N // grain) * grain
    e_pad = -(-E // grain) * grain

    x_pad = jnp.pad(transformed_features, ((0, n_pad - N), (0, 0)))
    src2 = jnp.pad(jnp.stack([edge_index_0[0], edge_index_1[0]]),
                   ((0, 0), (0, e_pad - E)))
    dst2 = jnp.pad(jnp.stack([edge_index_0[1], edge_index_1[1]]),
                   ((0, 0), (0, e_pad - E)), constant_values=N)
    et2 = jnp.pad(jnp.stack([edge_type_0, edge_type_1]),
                  ((0, 0), (0, e_pad - E)))
    fi2 = jnp.pad(jnp.stack([feature_index_0, feature_index_1]),
                  ((0, 0), (0, n_pad - N)))
    ni2 = jnp.pad(jnp.stack([new_index_0, new_index_1]),
                  ((0, 0), (0, n_pad - N)))
    zrows = jnp.zeros((n_pad // _NTILES, H), jnp.float32)

    table = _build_table(x_pad, W_rel, b_rel.reshape(R, 1, H))
    agg, deg = _sc_edges(table, src2, dst2, et2, fi2, zrows)
    Z = _sc_finalize(agg, deg, x_pad, fi2, ni2)
    Z0 = Z[0, :N]
    Z1 = Z[1, :N]
    scores, c01 = _fusion_stats(Z0, Z1, W_att, b_att.reshape(1, -1),
                                q_att.reshape(1, -1))
    fused, loss = _fusion_out(Z0, Z1, scores, c01, W_dec)
    return fused, loss[0, 0]


# speedup vs baseline: 1.0746x; 1.0150x over previous
import functools

import jax
import jax.numpy as jnp
from jax import lax
from jax.experimental import pallas as pl
from jax.experimental.pallas import tpu as pltpu
from jax.experimental.pallas import tpu_sc as plsc

_LANES = 16
_NTILES = 16
_NCORES = 2
_CH = 128


def _build_table(x_pad, W_rel, b_rel3):
    n_pad, H = x_pad.shape
    R = W_rel.shape[0]
    BLK = 1024
    nb = n_pad // BLK

    def body(x_ref, w_ref, b_ref, o_ref):
        mm = jnp.dot(x_ref[...], w_ref[0], preferred_element_type=jnp.float32)
        o_ref[0] = mm + b_ref[0, 0][None, :]

    out = pl.pallas_call(
        body,
        grid=(R, nb),
        in_specs=[
            pl.BlockSpec((BLK, H), lambda r, b: (b, 0)),
            pl.BlockSpec((1, H, H), lambda r, b: (r, 0, 0)),
            pl.BlockSpec((1, 1, H), lambda r, b: (r, 0, 0)),
        ],
        out_specs=pl.BlockSpec((1, BLK, H), lambda r, b: (r, b, 0)),
        out_shape=jax.ShapeDtypeStruct((R, n_pad, H), jnp.float32),
    )(x_pad, W_rel, b_rel3)
    return out.reshape(R * n_pad, H)


def _sc_edges(table, src2, dst2, et2, fi2, zrows):
    n_pad = fi2.shape[1]
    H = table.shape[1]
    e_pad = src2.shape[1]
    epw = e_pad // _NTILES
    nchunks = epw // _CH
    rpw = n_pad // _NTILES
    ndrows = n_pad // _CH
    mesh = plsc.VectorSubcoreMesh(core_axis_name="c", subcore_axis_name="s")

    @functools.partial(
        pl.kernel,
        out_type=(
            jax.ShapeDtypeStruct((_NCORES * n_pad, H), jnp.float32),
            jax.ShapeDtypeStruct((_NCORES, ndrows, _CH), jnp.float32),
        ),
        mesh=mesh,
        compiler_params=pltpu.CompilerParams(needs_layout_passes=False),
        scratch_types=[
            pltpu.VMEM_SHARED((n_pad, H), jnp.float32),
            pltpu.VMEM_SHARED((ndrows, _CH), jnp.float32),
            pltpu.VMEM((n_pad,), jnp.int32),
            pltpu.VMEM((_CH,), jnp.int32),
            pltpu.VMEM((_CH,), jnp.int32),
            pltpu.VMEM((1, _CH), jnp.int32),
            pltpu.VMEM((1, _CH), jnp.int32),
            pltpu.VMEM((_CH, H), jnp.float32),
            pltpu.VMEM((ndrows, _CH), jnp.float32),
            pltpu.VMEM((ndrows,), jnp.int32),
            pltpu.SemaphoreType.DMA,
        ],
    )
    def k(table_h, src_h, dst_h, et_h, fi_h, z_h,
          agg_h, deg_h,
          acc, deg_sh, fi_v, src_c, et_c, dst_c, gix_c, rows,
          degp, drow_ids, sem):
        c = lax.axis_index("c")
        s = lax.axis_index("s")
        pltpu.sync_copy(fi_h.at[c], fi_v)
        row0 = pl.multiple_of(s * rpw, 8)
        pltpu.sync_copy(z_h, acc.at[pl.ds(row0, rpw)])
        pltpu.sync_copy(z_h.at[pl.ds(0, ndrows)], degp)

        @pl.when(s == 0)
        def _():
            pltpu.sync_copy(z_h.at[pl.ds(0, ndrows)], deg_sh)

        for g in range(ndrows // _LANES):
            drow_ids[pl.ds(g * _LANES, _LANES)] = (
                lax.iota(jnp.int32, _LANES) + g * _LANES)
        plsc.subcore_barrier()

        def chunk_body(kk, carry):
            base = pl.multiple_of(s * epw + kk * _CH, 8)
            pltpu.sync_copy(src_h.at[c, pl.ds(base, _CH)], src_c)
            pltpu.sync_copy(et_h.at[c, pl.ds(base, _CH)], et_c)
            pltpu.sync_copy(dst_h.at[c, pl.ds(base, _CH)], dst_c.at[0])
            ones16 = jnp.ones((_LANES,), jnp.float32)
            for g in range(_CH // _LANES):
                sl = pl.ds(g * _LANES, _LANES)
                f16 = plsc.load_gather(fi_v, [src_c[sl]])
                gix_c[0, sl] = et_c[sl] * n_pad + f16
                d16 = dst_c[0, sl]
                plsc.addupdate_scatter(
                    degp, [lax.shift_right_logical(d16, 7),
                           jnp.bitwise_and(d16, _CH - 1)], ones16)
            pltpu.async_copy(table_h.at[gix_c.at[0]], rows, sem).wait()
            pltpu.sync_copy(rows, acc.at[dst_c.at[0]], add=True)
            return carry

        lax.fori_loop(0, nchunks, chunk_body, 0)
        pltpu.sync_copy(degp, deg_sh.at[drow_ids], add=True)
        plsc.subcore_barrier()

        pltpu.sync_copy(acc.at[pl.ds(row0, rpw)],
                        agg_h.at[pl.ds(pl.multiple_of(c * n_pad + row0, 8), rpw)])

        @pl.when(s == 0)
        def _():
            pltpu.sync_copy(deg_sh, deg_h.at[c])

    return k(table, src2, dst2, et2, fi2, zrows)


def _sc_finalize(agg, deg, x_pad, fi2, ni2):
    n_pad, H = x_pad.shape
    ndrows = n_pad // _CH
    rpw = n_pad // _NTILES
    nrchunks = rpw // _CH
    mesh = plsc.VectorSubcoreMesh(core_axis_name="c", subcore_axis_name="s")

    @functools.partial(
        pl.kernel,
        out_type=jax.ShapeDtypeStruct((_NCORES, n_pad, H), jnp.float32),
        mesh=mesh,
        compiler_params=pltpu.CompilerParams(needs_layout_passes=False),
        scratch_types=[
            pltpu.VMEM((n_pad,), jnp.int32),
            pltpu.VMEM((ndrows, _CH), jnp.float32),
            pltpu.VMEM((2, _CH), jnp.int32),
            pltpu.VMEM((2, _CH), jnp.int32),
            pltpu.VMEM((2, _CH), jnp.int32),
            pltpu.VMEM((2, _CH), jnp.float32),
            pltpu.VMEM((_CH, H), jnp.float32),
            pltpu.VMEM((_CH, H), jnp.float32),
            pltpu.VMEM((_CH, H), jnp.float32),
            pltpu.VMEM((_CH, H), jnp.float32),
            pltpu.VMEM((_CH, H), jnp.float32),
            pltpu.SemaphoreType.DMA,
            pltpu.SemaphoreType.DMA,
            pltpu.SemaphoreType.DMA,
            pltpu.SemaphoreType.DMA,
        ],
    )
    def k(agg_h, deg_h, x_h, fi_h, ni_h,
          out_h,
          fi_v, degv, j_c, gix_c, fic, dinv_c, rows0, rows1, xrows0, xrows1,
          orows, gsem0, gsem1, xsem0, xsem1):
        c = lax.axis_index("c")
        s = lax.axis_index("s")
        pltpu.sync_copy(fi_h.at[c], fi_v)
        pltpu.sync_copy(deg_h.at[c], degv)
        rbufs = (rows0, rows1)
        xbufs = (xrows0, xrows1)
        gsems = (gsem0, gsem1)
        xsems = (xsem0, xsem1)

        def prep(kk, i):
            base = pl.multiple_of(s * rpw + kk * _CH, 8)
            pltpu.sync_copy(ni_h.at[c, pl.ds(base, _CH)], j_c.at[i])
            for g in range(_CH // _LANES):
                sl = pl.ds(g * _LANES, _LANES)
                j16 = j_c[i, sl]
                fic[i, sl] = plsc.load_gather(fi_v, [j16])
                gix_c[i, sl] = j16 + c * n_pad
                d16 = plsc.load_gather(
                    degv, [lax.shift_right_logical(j16, 7),
                           jnp.bitwise_and(j16, _CH - 1)])
                dinv_c[i, sl] = 1.0 / jnp.maximum(d16, 1.0)
            ga = pltpu.async_copy(agg_h.at[gix_c.at[i]], rbufs[i], gsems[i])
            gx = pltpu.async_copy(x_h.at[fic.at[i]], xbufs[i], xsems[i])
            return ga, gx

        ds = [None, None]
        ds[0] = prep(0, 0)
        for kk in range(nrchunks):
            i = kk & 1
            if kk + 1 < nrchunks:
                ds[(kk + 1) & 1] = prep(kk + 1, (kk + 1) & 1)
            ds[i][0].wait()
            ds[i][1].wait()
            rows = rbufs[i]
            xrows = xbufs[i]

            def grp_body(g, rc, i=i, rows=rows, xrows=xrows):
                dv16 = dinv_c[i, pl.ds(pl.multiple_of(g * _LANES, _LANES),
                                       _LANES)]
                for lane in range(_LANES):
                    r = g * _LANES + lane
                    dv = dv16[lane]
                    for cg in range(H // _LANES):
                        slg = pl.ds(cg * _LANES, _LANES)
                        v = rows[r, slg] * dv + xrows[r, slg]
                        orows[r, slg] = jnp.maximum(v, 0.0)
                return rc

            lax.fori_loop(0, _CH // _LANES, grp_body, 0)
            base = pl.multiple_of(s * rpw + kk * _CH, 8)
            pltpu.sync_copy(orows, out_h.at[c, pl.ds(base, _CH)])

    return k(agg, deg, x_pad, fi2, ni2)


def _fusion_stats(Z0, Z1, W_att, b_att2, q_att2):
    Nn, H = Z0.shape
    I = W_att.shape[1]
    BLK = 2000
    nb = Nn // BLK

    def body(z0_ref, z1_ref, wa_ref, ba_ref, qa_ref, sc_ref, c_ref):
        step = pl.program_id(0)
        a0 = jnp.dot(z0_ref[...], wa_ref[...], preferred_element_type=jnp.float32)
        a1 = jnp.dot(z1_ref[...], wa_ref[...], preferred_element_type=jnp.float32)
        t0 = jnp.tanh(a0 + ba_ref[0][None, :])
        t1 = jnp.tanh(a1 + ba_ref[0][None, :])
        s0 = jnp.sum(t0 * qa_ref[0][None, :])
        s1 = jnp.sum(t1 * qa_ref[0][None, :])
        c01 = lax.dot_general(a0, a1, (((0,), (0,)), ((), ())),
                              preferred_element_type=jnp.float32)
        rr = lax.broadcasted_iota(jnp.int32, (8, 128), 0)
        cc = lax.broadcasted_iota(jnp.int32, (8, 128), 1)
        upd = (jnp.where((rr == 0) & (cc == 0), s0, 0.0)
               + jnp.where((rr == 0) & (cc == 1), s1, 0.0))

        @pl.when(step == 0)
        def _():
            sc_ref[...] = jnp.zeros_like(sc_ref)
            c_ref[...] = jnp.zeros_like(c_ref)

        sc_ref[...] += upd
        c_ref[...] += c01

    return pl.pallas_call(
        body,
        grid=(nb,),
        in_specs=[
            pl.BlockSpec((BLK, H), lambda b: (b, 0)),
            pl.BlockSpec((BLK, H), lambda b: (b, 0)),
            pl.BlockSpec((H, I), lambda b: (0, 0)),
            pl.BlockSpec((1, I), lambda b: (0, 0)),
            pl.BlockSpec((1, I), lambda b: (0, 0)),
        ],
        out_specs=[
            pl.BlockSpec((8, 128), lambda b: (0, 0)),
            pl.BlockSpec((I, I), lambda b: (0, 0)),
        ],
        out_shape=[
            jax.ShapeDtypeStruct((8, 128), jnp.float32),
            jax.ShapeDtypeStruct((I, I), jnp.float32),
        ],
    )(Z0, Z1, W_att, b_att2, q_att2)


def _fusion_out(Z0, Z1, scores, c01, W_dec):
    Nn, H = Z0.shape
    I = c01.shape[0]
    BLK = 2000
    nb = Nn // BLK

    def body(z0_ref, z1_ref, sc_ref, c_ref, wd_ref, f_ref, l_ref, acc_ref):
        step = pl.program_id(0)
        scv = sc_ref[...]
        rr = lax.broadcasted_iota(jnp.int32, (8, 128), 0)
        cc = lax.broadcasted_iota(jnp.int32, (8, 128), 1)
        s0 = jnp.sum(jnp.where((rr == 0) & (cc == 0), scv, 0.0)) / Nn
        s1 = jnp.sum(jnp.where((rr == 0) & (cc == 1), scv, 0.0)) / Nn
        m = jnp.maximum(s0, s1)
        e0 = jnp.exp(s0 - m)
        e1 = jnp.exp(s1 - m)
        a0 = e0 / (e0 + e1)
        a1 = e1 / (e0 + e1)
        z0 = z0_ref[...]
        z1 = z1_ref[...]
        fused = a0 * z0 + a1 * z1
        f_ref[...] = fused
        r0 = jnp.dot(fused, wd_ref[0], preferred_element_type=jnp.float32)
        r1 = jnp.dot(fused, wd_ref[1], preferred_element_type=jnp.float32)
        se = jnp.sum((r0 - z0) ** 2) + jnp.sum((r1 - z1) ** 2)

        @pl.when(step == 0)
        def _():
            acc_ref[0] = 0.0

        acc_ref[0] += se

        @pl.when(step == nb - 1)
        def _():
            cmat = c_ref[...] / Nn
            ortho = jnp.sum(cmat * cmat)
            l_ref[...] = jnp.full((1, 1), acc_ref[0] / (2.0 * Nn * H) + ortho,
                                  jnp.float32)

    return pl.pallas_call(
        body,
        grid=(nb,),
        in_specs=[
            pl.BlockSpec((BLK, H), lambda b: (b, 0)),
            pl.BlockSpec((BLK, H), lambda b: (b, 0)),
            pl.BlockSpec((8, 128), lambda b: (0, 0)),
            pl.BlockSpec((I, I), lambda b: (0, 0)),
            pl.BlockSpec((2, H, H), lambda b: (0, 0, 0)),
        ],
        out_specs=[
            pl.BlockSpec((BLK, H), lambda b: (b, 0)),
            pl.BlockSpec((1, 1), lambda b: (0, 0)),
        ],
        out_shape=[
            jax.ShapeDtypeStruct((Nn, H), jnp.float32),
            jax.ShapeDtypeStruct((1, 1), jnp.float32),
        ],
        scratch_shapes=[pltpu.SMEM((1,), jnp.float32)],
    )(Z0, Z1, scores, c01, W_dec)


def kernel(transformed_features, edge_index_0, edge_type_0, new_index_0,
           feature_index_0, edge_index_1, edge_type_1, new_index_1,
           feature_index_1, W_rel, b_rel, W_att, b_att, q_att, W_dec):
    N, H = transformed_features.shape
    E = edge_type_0.shape[0]
    R = W_rel.shape[0]
    grain = _NTILES * _CH
    n_pad = -(-N // grain) * grain
    e_pad = -(-E // grain) * grain

    x_pad = jnp.pad(transformed_features, ((0, n_pad - N), (0, 0)))
    src2 = jnp.pad(jnp.stack([edge_index_0[0], edge_index_1[0]]),
                   ((0, 0), (0, e_pad - E)))
    dst2 = jnp.pad(jnp.stack([edge_index_0[1], edge_index_1[1]]),
                   ((0, 0), (0, e_pad - E)), constant_values=N)
    et2 = jnp.pad(jnp.stack([edge_type_0, edge_type_1]),
                  ((0, 0), (0, e_pad - E)))
    fi2 = jnp.pad(jnp.stack([feature_index_0, feature_index_1]),
                  ((0, 0), (0, n_pad - N)))
    ni2 = jnp.pad(jnp.stack([new_index_0, new_index_1]),
                  ((0, 0), (0, n_pad - N)))
    zrows = jnp.zeros((n_pad // _NTILES, H), jnp.float32)

    table = _build_table(x_pad, W_rel, b_rel.reshape(R, 1, H))
    agg, deg = _sc_edges(table, src2, dst2, et2, fi2, zrows)
    Z = _sc_finalize(agg, deg, x_pad, fi2, ni2)
    Z0 = Z[0, :N]
    Z1 = Z[1, :N]
    scores, c01 = _fusion_stats(Z0, Z1, W_att, b_att.reshape(1, -1),
                                q_att.reshape(1, -1))
    fused, loss = _fusion_out(Z0, Z1, scores, c01, W_dec)
    return fused, loss[0, 0]


# speedup vs baseline: 1.0788x; 1.0039x over previous
import functools

import jax
import jax.numpy as jnp
from jax import lax
from jax.experimental import pallas as pl
from jax.experimental.pallas import tpu as pltpu
from jax.experimental.pallas import tpu_sc as plsc

_LANES = 16
_NTILES = 16
_NCORES = 2
_CH = 128


def _build_table(x_pad, W_rel, b_rel3):
    n_pad, H = x_pad.shape
    R = W_rel.shape[0]
    BLK = 1024
    nb = n_pad // BLK

    def body(x_ref, w_ref, b_ref, o_ref):
        mm = jnp.dot(x_ref[...], w_ref[0], preferred_element_type=jnp.float32)
        o_ref[0] = mm + b_ref[0, 0][None, :]

    out = pl.pallas_call(
        body,
        grid=(R, nb),
        in_specs=[
            pl.BlockSpec((BLK, H), lambda r, b: (b, 0)),
            pl.BlockSpec((1, H, H), lambda r, b: (r, 0, 0)),
            pl.BlockSpec((1, 1, H), lambda r, b: (r, 0, 0)),
        ],
        out_specs=pl.BlockSpec((1, BLK, H), lambda r, b: (r, b, 0)),
        out_shape=jax.ShapeDtypeStruct((R, n_pad, H), jnp.float32),
    )(x_pad, W_rel, b_rel3)
    return out.reshape(R * n_pad, H)


def _sc_edges(table, src2, dst2, et2, fi2, zrows):
    n_pad = fi2.shape[1]
    H = table.shape[1]
    e_pad = src2.shape[1]
    epw = e_pad // _NTILES
    nchunks = epw // _CH
    rpw = n_pad // _NTILES
    ndrows = n_pad // _CH
    mesh = plsc.VectorSubcoreMesh(core_axis_name="c", subcore_axis_name="s")

    @functools.partial(
        pl.kernel,
        out_type=(
            jax.ShapeDtypeStruct((_NCORES * n_pad, H), jnp.float32),
            jax.ShapeDtypeStruct((_NCORES, ndrows, _CH), jnp.float32),
        ),
        mesh=mesh,
        compiler_params=pltpu.CompilerParams(needs_layout_passes=False),
        scratch_types=[
            pltpu.VMEM_SHARED((n_pad, H), jnp.float32),
            pltpu.VMEM_SHARED((ndrows, _CH), jnp.float32),
            pltpu.VMEM((n_pad,), jnp.int32),
            pltpu.VMEM((_CH,), jnp.int32),
            pltpu.VMEM((_CH,), jnp.int32),
            pltpu.VMEM((1, _CH), jnp.int32),
            pltpu.VMEM((1, _CH), jnp.int32),
            pltpu.VMEM((_CH, H), jnp.float32),
            pltpu.VMEM((ndrows, _CH), jnp.float32),
            pltpu.VMEM((ndrows,), jnp.int32),
            pltpu.SemaphoreType.DMA,
        ],
    )
    def k(table_h, src_h, dst_h, et_h, fi_h, z_h,
          agg_h, deg_h,
          acc, deg_sh, fi_v, src_c, et_c, dst_c, gix_c, rows,
          degp, drow_ids, sem):
        c = lax.axis_index("c")
        s = lax.axis_index("s")
        pltpu.sync_copy(fi_h.at[c], fi_v)
        row0 = pl.multiple_of(s * rpw, 8)
        pltpu.sync_copy(z_h, acc.at[pl.ds(row0, rpw)])
        pltpu.sync_copy(z_h.at[pl.ds(0, ndrows)], degp)

        @pl.when(s == 0)
        def _():
            pltpu.sync_copy(z_h.at[pl.ds(0, ndrows)], deg_sh)

        for g in range(ndrows // _LANES):
            drow_ids[pl.ds(g * _LANES, _LANES)] = (
                lax.iota(jnp.int32, _LANES) + g * _LANES)
        plsc.subcore_barrier()

        def chunk_body(kk, carry):
            base = pl.multiple_of(s * epw + kk * _CH, 8)
            pltpu.sync_copy(src_h.at[c, pl.ds(base, _CH)], src_c)
            pltpu.sync_copy(et_h.at[c, pl.ds(base, _CH)], et_c)
            pltpu.sync_copy(dst_h.at[c, pl.ds(base, _CH)], dst_c.at[0])
            ones16 = jnp.ones((_LANES,), jnp.float32)
            for g in range(_CH // _LANES):
                sl = pl.ds(g * _LANES, _LANES)
                f16 = plsc.load_gather(fi_v, [src_c[sl]])
                gix_c[0, sl] = et_c[sl] * n_pad + f16
                d16 = dst_c[0, sl]
                plsc.addupdate_scatter(
                    degp, [lax.shift_right_logical(d16, 7),
                           jnp.bitwise_and(d16, _CH - 1)], ones16)
            pltpu.async_copy(table_h.at[gix_c.at[0]], rows, sem).wait()
            pltpu.sync_copy(rows, acc.at[dst_c.at[0]], add=True)
            return carry

        lax.fori_loop(0, nchunks, chunk_body, 0)
        pltpu.sync_copy(degp, deg_sh.at[drow_ids], add=True)
        plsc.subcore_barrier()

        pltpu.sync_copy(acc.at[pl.ds(row0, rpw)],
                        agg_h.at[pl.ds(pl.multiple_of(c * n_pad + row0, 8), rpw)])

        @pl.when(s == 0)
        def _():
            pltpu.sync_copy(deg_sh, deg_h.at[c])

    return k(table, src2, dst2, et2, fi2, zrows)


def _sc_finalize(agg, deg, x_pad, fi2, ni2):
    n_pad, H = x_pad.shape
    ndrows = n_pad // _CH
    rpw = n_pad // _NTILES
    nrchunks = rpw // _CH
    mesh = plsc.VectorSubcoreMesh(core_axis_name="c", subcore_axis_name="s")

    @functools.partial(
        pl.kernel,
        out_type=jax.ShapeDtypeStruct((_NCORES, n_pad, H), jnp.float32),
        mesh=mesh,
        compiler_params=pltpu.CompilerParams(needs_layout_passes=False),
        scratch_types=[
            pltpu.VMEM((n_pad,), jnp.int32),
            pltpu.VMEM((ndrows, _CH), jnp.float32),
            pltpu.VMEM((2, _CH), jnp.int32),
            pltpu.VMEM((2, _CH), jnp.int32),
            pltpu.VMEM((2, _CH), jnp.int32),
            pltpu.VMEM((2, _CH), jnp.float32),
            pltpu.VMEM((_CH, H), jnp.float32),
            pltpu.VMEM((_CH, H), jnp.float32),
            pltpu.VMEM((_CH, H), jnp.float32),
            pltpu.VMEM((_CH, H), jnp.float32),
            pltpu.VMEM((_CH, H), jnp.float32),
            pltpu.SemaphoreType.DMA,
            pltpu.SemaphoreType.DMA,
            pltpu.SemaphoreType.DMA,
            pltpu.SemaphoreType.DMA,
        ],
    )
    def k(agg_h, deg_h, x_h, fi_h, ni_h,
          out_h,
          fi_v, degv, j_c, gix_c, fic, dinv_c, rows0, rows1, xrows0, xrows1,
          orows, gsem0, gsem1, xsem0, xsem1):
        c = lax.axis_index("c")
        s = lax.axis_index("s")
        pltpu.sync_copy(fi_h.at[c], fi_v)
        pltpu.sync_copy(deg_h.at[c], degv)
        rbufs = (rows0, rows1)
        xbufs = (xrows0, xrows1)
        gsems = (gsem0, gsem1)
        xsems = (xsem0, xsem1)

        def prep(kk, i):
            base = pl.multiple_of(s * rpw + kk * _CH, 8)
            pltpu.sync_copy(ni_h.at[c, pl.ds(base, _CH)], j_c.at[i])
            for g in range(_CH // _LANES):
                sl = pl.ds(g * _LANES, _LANES)
                j16 = j_c[i, sl]
                fic[i, sl] = plsc.load_gather(fi_v, [j16])
                gix_c[i, sl] = j16 + c * n_pad
                d16 = plsc.load_gather(
                    degv, [lax.shift_right_logical(j16, 7),
                           jnp.bitwise_and(j16, _CH - 1)])
                dinv_c[i, sl] = 1.0 / jnp.maximum(d16, 1.0)
            ga = pltpu.async_copy(agg_h.at[gix_c.at[i]], rbufs[i], gsems[i])
            gx = pltpu.async_copy(x_h.at[fic.at[i]], xbufs[i], xsems[i])
            return ga, gx

        ds = [None, None]
        ds[0] = prep(0, 0)
        for kk in range(nrchunks):
            i = kk & 1
            if kk + 1 < nrchunks:
                ds[(kk + 1) & 1] = prep(kk + 1, (kk + 1) & 1)
            ds[i][0].wait()
            ds[i][1].wait()
            rows = rbufs[i]
            xrows = xbufs[i]

            def grp_body(g, rc, i=i, rows=rows, xrows=xrows):
                dv16 = dinv_c[i, pl.ds(pl.multiple_of(g * _LANES, _LANES),
                                       _LANES)]
                for lane in range(_LANES):
                    r = g * _LANES + lane
                    dv = dv16[lane]
                    for cg in range(H // _LANES):
                        slg = pl.ds(cg * _LANES, _LANES)
                        v = rows[r, slg] * dv + xrows[r, slg]
                        orows[r, slg] = jnp.maximum(v, 0.0)
                return rc

            lax.fori_loop(0, _CH // _LANES, grp_body, 0)
            base = pl.multiple_of(s * rpw + kk * _CH, 8)
            pltpu.sync_copy(orows, out_h.at[c, pl.ds(base, _CH)])

    return k(agg, deg, x_pad, fi2, ni2)


def _fusion(Z0, Z1, W_att, b_att2, q_att2, W_dec):
    Nn, H = Z0.shape
    I = W_att.shape[1]
    BLK = 2000
    nb = Nn // BLK

    def body(z0_ref, z1_ref, wa_ref, ba_ref, qa_ref, wd_ref, f_ref, l_ref,
             sc_ref, c_ref, acc_ref):
        step = pl.program_id(0)
        rr = lax.broadcasted_iota(jnp.int32, (8, 128), 0)
        cc = lax.broadcasted_iota(jnp.int32, (8, 128), 1)
        z0 = z0_ref[...]
        z1 = z1_ref[...]

        @pl.when(step == 0)
        def _():
            sc_ref[...] = jnp.zeros_like(sc_ref)
            c_ref[...] = jnp.zeros_like(c_ref)
            acc_ref[0] = 0.0

        @pl.when(step < nb)
        def _():
            a0 = jnp.dot(z0, wa_ref[...], preferred_element_type=jnp.float32)
            a1 = jnp.dot(z1, wa_ref[...], preferred_element_type=jnp.float32)
            t0 = jnp.tanh(a0 + ba_ref[0][None, :])
            t1 = jnp.tanh(a1 + ba_ref[0][None, :])
            s0 = jnp.sum(t0 * qa_ref[0][None, :])
            s1 = jnp.sum(t1 * qa_ref[0][None, :])
            c01 = lax.dot_general(a0, a1, (((0,), (0,)), ((), ())),
                                  preferred_element_type=jnp.float32)
            upd = (jnp.where((rr == 0) & (cc == 0), s0, 0.0)
                   + jnp.where((rr == 0) & (cc == 1), s1, 0.0))
            sc_ref[...] += upd
            c_ref[...] += c01

        @pl.when(step >= nb)
        def _():
            scv = sc_ref[...]
            s0 = jnp.sum(jnp.where((rr == 0) & (cc == 0), scv, 0.0)) / Nn
            s1 = jnp.sum(jnp.where((rr == 0) & (cc == 1), scv, 0.0)) / Nn
            m = jnp.maximum(s0, s1)
            e0 = jnp.exp(s0 - m)
            e1 = jnp.exp(s1 - m)
            a0 = e0 / (e0 + e1)
            a1 = e1 / (e0 + e1)
            fused = a0 * z0 + a1 * z1
            f_ref[...] = fused
            r0 = jnp.dot(fused, wd_ref[0], preferred_element_type=jnp.float32)
            r1 = jnp.dot(fused, wd_ref[1], preferred_element_type=jnp.float32)
            acc_ref[0] += jnp.sum((r0 - z0) ** 2) + jnp.sum((r1 - z1) ** 2)

            @pl.when(step == 2 * nb - 1)
            def _():
                cmat = c_ref[...] / Nn
                ortho = jnp.sum(cmat * cmat)
                l_ref[...] = jnp.full(
                    (1, 1), acc_ref[0] / (2.0 * Nn * H) + ortho, jnp.float32)

    return pl.pallas_call(
        body,
        grid=(2 * nb,),
        in_specs=[
            pl.BlockSpec((BLK, H), lambda b: (b % nb, 0)),
            pl.BlockSpec((BLK, H), lambda b: (b % nb, 0)),
            pl.BlockSpec((H, I), lambda b: (0, 0)),
            pl.BlockSpec((1, I), lambda b: (0, 0)),
            pl.BlockSpec((1, I), lambda b: (0, 0)),
            pl.BlockSpec((2, H, H), lambda b: (0, 0, 0)),
        ],
        out_specs=[
            pl.BlockSpec((BLK, H), lambda b: (b % nb, 0)),
            pl.BlockSpec((1, 1), lambda b: (0, 0)),
        ],
        out_shape=[
            jax.ShapeDtypeStruct((Nn, H), jnp.float32),
            jax.ShapeDtypeStruct((1, 1), jnp.float32),
        ],
        scratch_shapes=[
            pltpu.VMEM((8, 128), jnp.float32),
            pltpu.VMEM((I, I), jnp.float32),
            pltpu.SMEM((1,), jnp.float32),
        ],
    )(Z0, Z1, W_att, b_att2, q_att2, W_dec)


def kernel(transformed_features, edge_index_0, edge_type_0, new_index_0,
           feature_index_0, edge_index_1, edge_type_1, new_index_1,
           feature_index_1, W_rel, b_rel, W_att, b_att, q_att, W_dec):
    N, H = transformed_features.shape
    E = edge_type_0.shape[0]
    R = W_rel.shape[0]
    grain = _NTILES * _CH
    n_pad = -(-N // grain) * grain
    e_pad = -(-E // grain) * grain

    x_pad = jnp.pad(transformed_features, ((0, n_pad - N), (0, 0)))
    src2 = jnp.pad(jnp.stack([edge_index_0[0], edge_index_1[0]]),
                   ((0, 0), (0, e_pad - E)))
    dst2 = jnp.pad(jnp.stack([edge_index_0[1], edge_index_1[1]]),
                   ((0, 0), (0, e_pad - E)), constant_values=N)
    et2 = jnp.pad(jnp.stack([edge_type_0, edge_type_1]),
                  ((0, 0), (0, e_pad - E)))
    fi2 = jnp.pad(jnp.stack([feature_index_0, feature_index_1]),
                  ((0, 0), (0, n_pad - N)))
    ni2 = jnp.pad(jnp.stack([new_index_0, new_index_1]),
                  ((0, 0), (0, n_pad - N)))
    zrows = jnp.zeros((n_pad // _NTILES, H), jnp.float32)

    table = _build_table(x_pad, W_rel, b_rel.reshape(R, 1, H))
    agg, deg = _sc_edges(table, src2, dst2, et2, fi2, zrows)
    Z = _sc_finalize(agg, deg, x_pad, fi2, ni2)
    Z0 = Z[0, :N]
    Z1 = Z[1, :N]
    fused, loss = _fusion(Z0, Z1, W_att, b_att.reshape(1, -1),
                          q_att.reshape(1, -1), W_dec)
    return fused, loss[0, 0]
